# baseline (speedup 1.0000x reference)
import functools

import jax
import jax.numpy as jnp
from jax import lax
from jax.experimental import pallas as pl
from jax.experimental.pallas import tpu as pltpu
from jax.experimental.pallas import tpu_sc as plsc

_N = 16384
_H = 64
_D = 128
_F = 3
_V2 = 50000

_NC = 2
_NS = 16
_NW = _NC * _NS
_BPW = _N // _NW
_CL = 128
_KCH = _BPW // _CL

_BT = 1024


def _sc_gather(act2, res2, aidx, ridx):
    mesh = plsc.VectorSubcoreMesh(core_axis_name="c", subcore_axis_name="s")

    @functools.partial(
        pl.kernel,
        mesh=mesh,
        out_type=[
            jax.ShapeDtypeStruct((_N, _D), jnp.float32),
            jax.ShapeDtypeStruct((_N, _D), jnp.float32),
        ],
        scratch_types=[
            pltpu.VMEM((_BPW,), jnp.int32),
            pltpu.VMEM((_BPW,), jnp.int32),
            pltpu.VMEM((2, _CL, _D), jnp.float32),
            pltpu.VMEM((2, _CL, _D), jnp.float32),
            pltpu.SemaphoreType.DMA,
            pltpu.SemaphoreType.DMA,
            pltpu.SemaphoreType.DMA,
            pltpu.SemaphoreType.DMA,
        ],
        compiler_params=pltpu.CompilerParams(use_tc_tiling_on_sc=False),
    )
    def gather_k(act_t, res_t, aidx_h, ridx_h, act_o, res_o,
                 aidx_v, ridx_v, abuf, rbuf, g0, g1, w0, w1):
        wid = lax.axis_index("s") * _NC + lax.axis_index("c")
        base = wid * _BPW
        pltpu.sync_copy(aidx_h.at[pl.ds(base, _BPW)], aidx_v)
        pltpu.sync_copy(ridx_h.at[pl.ds(base, _BPW)], ridx_v)
        gsem = [g0, g1]
        wsem = [w0, w1]

        def fire_gather(j):
            b = j % 2
            ix = pl.ds(j * _CL, _CL)
            return [
                pltpu.async_copy(act_t.at[aidx_v.at[ix]], abuf.at[b], gsem[b]),
                pltpu.async_copy(res_t.at[ridx_v.at[ix]], rbuf.at[b], gsem[b]),
            ]

        def fire_write(j):
            b = j % 2
            ox = pl.ds(base + j * _CL, _CL)
            return [
                pltpu.async_copy(abuf.at[b], act_o.at[ox], wsem[b]),
                pltpu.async_copy(rbuf.at[b], res_o.at[ox], wsem[b]),
            ]

        gd = {0: fire_gather(0)}
        wd = {}
        for j in range(_KCH):
            if j + 1 < _KCH:
                if j - 1 >= 0:
                    for c in wd[j - 1]:
                        c.wait()
                gd[j + 1] = fire_gather(j + 1)
            for c in gd[j]:
                c.wait()
            wd[j] = fire_write(j)
        for j in (_KCH - 2, _KCH - 1):
            for c in wd[j]:
                c.wait()

    return gather_k(act2, res2, aidx, ridx)


def _ln_rows(x, g, b, eps=1e-5):
    mu = jnp.mean(x, axis=-1, keepdims=True)
    var = jnp.mean((x - mu) ** 2, axis=-1, keepdims=True)
    return (x - mu) / jnp.sqrt(var + eps) * g + b


def _dense_body(a_ref, r_ref, nm_ref, ga_ref, gr_ref,
                nlg_ref, nlb_ref, w1_ref, b1_ref, mlg_ref, mlb_ref,
                wg_ref, bg_ref, wb_ref, bb_ref, wp_ref, bp_ref,
                plg_ref, plb_ref, out_ref):
    f32 = jnp.float32
    a = a_ref[...]
    r = r_ref[...]
    pa = (a & 1) == 1
    pr = (r & 1) == 1
    ga = ga_ref[...]
    gr = gr_ref[...]
    ah = jnp.where(pa, ga[:, _H:], ga[:, :_H])
    rh = jnp.where(pr, gr[:, _H:], gr[:, :_H])
    cat = jnp.concatenate([ah, rh], axis=-1)

    nf = jnp.log1p(jnp.maximum(nm_ref[...], 0.0))
    nf = _ln_rows(nf, nlg_ref[...], nlb_ref[...])
    h = jnp.maximum(
        jnp.dot(nf, w1_ref[...], preferred_element_type=f32) + b1_ref[...], 0.0)
    num_emb = _ln_rows(h, mlg_ref[...], mlb_ref[...])
    gamma = jax.nn.sigmoid(
        jnp.dot(num_emb, wg_ref[...], preferred_element_type=f32) + bg_ref[...])
    beta = jnp.dot(num_emb, wb_ref[...], preferred_element_type=f32) + bb_ref[...]
    cat_mod = cat * gamma + beta
    is_pad = (a == 0) & (r == 0)
    cat_mod = jnp.where(is_pad, 0.0, cat_mod)
    num_emb = jnp.where(is_pad, 0.0, num_emb)
    pre = (jnp.dot(cat_mod, wp_ref[0:_D, :], preferred_element_type=f32)
           + jnp.dot(num_emb, wp_ref[_D:2 * _D, :], preferred_element_type=f32)
           + bp_ref[...])
    out_ref[...] = _ln_rows(jnp.maximum(pre, 0.0), plg_ref[...], plb_ref[...])


def _tc_dense(acts2, ress2, num_arr, act_rows, res_rows,
              num_ln_g, num_ln_b, W1, b1, mlp_ln_g, mlp_ln_b,
              Wg, bg, Wb, bb, Wp, bp, proj_ln_g, proj_ln_b):
    grid = (_N // _BT,)
    row = lambda i: (i, 0)
    full1 = lambda i: (0,)
    full2 = lambda i: (0, 0)
    in_specs = [
        pl.BlockSpec((_BT, 1), row),
        pl.BlockSpec((_BT, 1), row),
        pl.BlockSpec((_BT, _F), row),
        pl.BlockSpec((_BT, _D), row),
        pl.BlockSpec((_BT, _D), row),
        pl.BlockSpec((_F,), full1),
        pl.BlockSpec((_F,), full1),
        pl.BlockSpec((_F, _D), full2),
        pl.BlockSpec((_D,), full1),
        pl.BlockSpec((_D,), full1),
        pl.BlockSpec((_D,), full1),
        pl.BlockSpec((_D, _D), full2),
        pl.BlockSpec((_D,), full1),
        pl.BlockSpec((_D, _D), full2),
        pl.BlockSpec((_D,), full1),
        pl.BlockSpec((2 * _D, _D), full2),
        pl.BlockSpec((_D,), full1),
        pl.BlockSpec((_D,), full1),
        pl.BlockSpec((_D,), full1),
    ]
    return pl.pallas_call(
        _dense_body,
        grid=grid,
        in_specs=in_specs,
        out_specs=pl.BlockSpec((_BT, _D), row),
        out_shape=jax.ShapeDtypeStruct((_N, _D), jnp.float32),
        compiler_params=pltpu.CompilerParams(
            dimension_semantics=("parallel",)),
    )(acts2, ress2, num_arr, act_rows, res_rows,
      num_ln_g, num_ln_b, W1, b1, mlp_ln_g, mlp_ln_b,
      Wg, bg, Wb, bb, Wp, bp, proj_ln_g, proj_ln_b)


def kernel(activities, resources, num_arr, act_table, res_table,
           num_ln_g, num_ln_b, W1, b1, mlp_ln_g, mlp_ln_b,
           Wg, bg, Wb, bb, Wp, bp, proj_ln_g, proj_ln_b):
    act2 = act_table.reshape(_V2, _D)
    res2 = res_table.reshape(_V2, _D)
    acts = activities.astype(jnp.int32)
    ress = resources.astype(jnp.int32)
    act_rows, res_rows = _sc_gather(act2, res2, acts >> 1, ress >> 1)
    return _tc_dense(acts.reshape(_N, 1), ress.reshape(_N, 1), num_arr,
                     act_rows, res_rows,
                     num_ln_g, num_ln_b, W1, b1, mlp_ln_g, mlp_ln_b,
                     Wg, bg, Wb, bb, Wp, bp, proj_ln_g, proj_ln_b)
